# Initial kernel scaffold; baseline (speedup 1.0000x reference)
import functools

import jax
import jax.numpy as jnp
from jax.experimental import pallas as pl
from jax.experimental.pallas import tpu as pltpu
from jax.experimental.pallas import tpu_sc as plsc

B = 16384
S = 50
D = 64
N = B * S

WIN = 128


def _emb_body(table_hbm, idx_vmem, out_vmem):
    pltpu.sync_copy(table_hbm.at[idx_vmem.at[0]], out_vmem)


def _kernel_body(hour_hbm, week_hbm, wh_hbm, ww_hbm, week_out, hour_out):
    for table, idx, out in ((wh_hbm, hour_hbm, hour_out),
                            (ww_hbm, week_hbm, week_out)):
        pltpu.emit_pipeline(
            functools.partial(_emb_body, table),
            grid=(N // WIN,),
            in_specs=[pl.BlockSpec((1, WIN), index_map=lambda i: (0, i))],
            out_specs=[pl.BlockSpec((WIN, D), index_map=lambda i: (i, 0))],
            core_axis_name=("core", "subcore"),
            dimension_semantics=(pltpu.PARALLEL,),
        )(idx, out)


def kernel(hour, week, W_hour, W_week):
    mesh = plsc.VectorSubcoreMesh(core_axis_name="core",
                                  subcore_axis_name="subcore")
    out_t = jax.ShapeDtypeStruct((N, D), jnp.float32)
    k = pl.kernel(_kernel_body, mesh=mesh, out_type=(out_t, out_t))
    week_e, hour_e = k(hour.reshape(1, N), week.reshape(1, N), W_hour, W_week)
    return (week_e.reshape(B, S, D), hour_e.reshape(B, S, D))


# speedup vs baseline: 1.2731x; 1.2731x over previous
import functools

import jax
import jax.numpy as jnp
from jax.experimental import pallas as pl
from jax.experimental.pallas import tpu as pltpu
from jax.experimental.pallas import tpu_sc as plsc

B = 16384
S = 50
D = 64
N = B * S

WIN = 128


def _emb_body(table_hbm, idx_vmem, out_vmem):
    pltpu.sync_copy(table_hbm.at[idx_vmem.at[0]], out_vmem)


def _kernel_body(hour_hbm, week_hbm, wh_hbm, ww_hbm, week_out, hour_out):
    for table, idx, out in ((wh_hbm, hour_hbm, hour_out),
                            (ww_hbm, week_hbm, week_out)):
        pltpu.emit_pipeline(
            functools.partial(_emb_body, table),
            grid=(N // WIN,),
            in_specs=[pl.BlockSpec((1, WIN), index_map=lambda i: (0, i))],
            out_specs=[pl.BlockSpec((WIN, D), index_map=lambda i: (i, 0))],
            core_axis_name=("core", "subcore"),
            dimension_semantics=(pltpu.PARALLEL,),
        )(idx, out)


def kernel(hour, week, W_hour, W_week):
    mesh = plsc.VectorSubcoreMesh(core_axis_name="core",
                                  subcore_axis_name="subcore")
    out_t = jax.ShapeDtypeStruct((N, D), jnp.float32)
    k = pl.kernel(_kernel_body, mesh=mesh, out_type=(out_t, out_t),
                  compiler_params=pltpu.CompilerParams(
                      use_tc_tiling_on_sc=False))
    week_e, hour_e = k(hour.reshape(1, N), week.reshape(1, N), W_hour, W_week)
    return (week_e.reshape(B, S, D), hour_e.reshape(B, S, D))


# speedup vs baseline: 1.2838x; 1.0084x over previous
import functools

import jax
import jax.numpy as jnp
from jax.experimental import pallas as pl
from jax.experimental.pallas import tpu as pltpu
from jax.experimental.pallas import tpu_sc as plsc

B = 16384
S = 50
D = 64
N = B * S

WIN = 512


def _emb_body(table_hbm, idx_vmem, out_vmem):
    pltpu.sync_copy(table_hbm.at[idx_vmem.at[0]], out_vmem)


def _kernel_body(hour_hbm, week_hbm, wh_hbm, ww_hbm, week_out, hour_out):
    for table, idx, out in ((wh_hbm, hour_hbm, hour_out),
                            (ww_hbm, week_hbm, week_out)):
        pltpu.emit_pipeline(
            functools.partial(_emb_body, table),
            grid=(N // WIN,),
            in_specs=[pl.BlockSpec((1, WIN), index_map=lambda i: (0, i))],
            out_specs=[pl.BlockSpec((WIN, D), index_map=lambda i: (i, 0))],
            core_axis_name=("core", "subcore"),
            dimension_semantics=(pltpu.PARALLEL,),
        )(idx, out)


def kernel(hour, week, W_hour, W_week):
    mesh = plsc.VectorSubcoreMesh(core_axis_name="core",
                                  subcore_axis_name="subcore")
    out_t = jax.ShapeDtypeStruct((N, D), jnp.float32)
    k = pl.kernel(_kernel_body, mesh=mesh, out_type=(out_t, out_t),
                  compiler_params=pltpu.CompilerParams(
                      use_tc_tiling_on_sc=False))
    week_e, hour_e = k(hour.reshape(1, N), week.reshape(1, N), W_hour, W_week)
    return (week_e.reshape(B, S, D), hour_e.reshape(B, S, D))


# speedup vs baseline: 7.5779x; 5.9028x over previous
import functools

import jax
import jax.numpy as jnp
from jax import lax
from jax.experimental import pallas as pl
from jax.experimental.pallas import tpu as pltpu
from jax.experimental.pallas import tpu_sc as plsc

B = 16384
S = 50
D = 64
N = B * S

WIN = 512


def _emb_body(table_sp, idx_vmem, out_vmem):
    pltpu.sync_copy(table_sp.at[idx_vmem.at[0]], out_vmem)


def _kernel_body(hour_hbm, week_hbm, wh_hbm, ww_hbm, week_out, hour_out,
                 wh_sp, ww_sp):
    sid = lax.axis_index("subcore")

    @pl.when(sid == 0)
    def _stage():
        pltpu.sync_copy(wh_hbm, wh_sp)
        pltpu.sync_copy(ww_hbm, ww_sp)

    plsc.subcore_barrier()

    for table, idx, out in ((wh_sp, hour_hbm, hour_out),
                            (ww_sp, week_hbm, week_out)):
        pltpu.emit_pipeline(
            functools.partial(_emb_body, table),
            grid=(N // WIN,),
            in_specs=[pl.BlockSpec((1, WIN), index_map=lambda i: (0, i))],
            out_specs=[pl.BlockSpec((WIN, D), index_map=lambda i: (i, 0))],
            core_axis_name=("core", "subcore"),
            dimension_semantics=(pltpu.PARALLEL,),
        )(idx, out)


def kernel(hour, week, W_hour, W_week):
    mesh = plsc.VectorSubcoreMesh(core_axis_name="core",
                                  subcore_axis_name="subcore")
    out_t = jax.ShapeDtypeStruct((N, D), jnp.float32)
    k = pl.kernel(_kernel_body, mesh=mesh, out_type=(out_t, out_t),
                  scratch_types=[
                      pltpu.VMEM_SHARED((25, D), jnp.float32),
                      pltpu.VMEM_SHARED((8, D), jnp.float32),
                  ],
                  compiler_params=pltpu.CompilerParams(
                      use_tc_tiling_on_sc=False))
    week_e, hour_e = k(hour.reshape(1, N), week.reshape(1, N), W_hour, W_week)
    return (week_e.reshape(B, S, D), hour_e.reshape(B, S, D))
